# Initial kernel scaffold; baseline (speedup 1.0000x reference)
import jax
import jax.numpy as jnp
from jax.experimental import pallas as pl


def kernel(velocities, distance_matrix, W1, b1, gamma, beta, W2, b2):
    raise NotImplementedError("write your pallas kernel here")


# speedup vs baseline: 2.2180x; 2.2180x over previous
import functools

import jax
import jax.numpy as jnp
from jax import lax
from jax.experimental import pallas as pl
from jax.experimental.pallas import tpu as pltpu
from jax.experimental.pallas import tpu_sc as plsc

B, A, D = 16, 128, 512
NC, NS, L = 2, 16, 16
NW = NC * NS
ROWS = B * A
RPW = ROWS // NW
NG = RPW // L
CW = 8
MR = 256


@functools.partial(
    pl.kernel,
    out_type=jax.ShapeDtypeStruct((ROWS * CW,), jnp.float32),
    mesh=plsc.VectorSubcoreMesh(core_axis_name="c", subcore_axis_name="s"),
    compiler_params=pltpu.CompilerParams(needs_layout_passes=False),
    scratch_types=[
        pltpu.VMEM((RPW * A,), jnp.float32),
        pltpu.VMEM((A * 3,), jnp.float32),
        pltpu.VMEM((RPW * CW,), jnp.float32),
    ],
)
def _sc_neighbors(d_hbm, v_hbm, out_hbm, dblk, vblk, oblk):
    wid = lax.axis_index("s") * NC + lax.axis_index("c")
    row0 = wid * RPW
    b = row0 // A
    pltpu.sync_copy(d_hbm.at[pl.ds(row0 * A, RPW * A)], dblk)
    pltpu.sync_copy(v_hbm.at[pl.ds(b * A * 3, A * 3)], vblk)

    iota = lax.iota(jnp.int32, L)
    inf = jnp.full((L,), jnp.inf, jnp.float32)
    zi = jnp.zeros((L,), jnp.int32)
    zero = jnp.zeros((L,), jnp.float32)

    for g in range(NG):
        rows = g * L + iota

        def col_body(j, carry, rows=rows):
            m1, m2, m3, m4, i1, i2, i3, i4 = carry
            jv = jnp.full((L,), j, jnp.int32)
            dj = plsc.load_gather(dblk, [rows * A + jv])
            c1 = dj < m1; c2 = dj < m2; c3 = dj < m3; c4 = dj < m4
            nm4 = jnp.where(c4, jnp.where(c3, m3, dj), m4)
            ni4 = jnp.where(c4, jnp.where(c3, i3, jv), i4)
            nm3 = jnp.where(c3, jnp.where(c2, m2, dj), m3)
            ni3 = jnp.where(c3, jnp.where(c2, i2, jv), i3)
            nm2 = jnp.where(c2, jnp.where(c1, m1, dj), m2)
            ni2 = jnp.where(c2, jnp.where(c1, i1, jv), i2)
            nm1 = jnp.where(c1, dj, m1)
            ni1 = jnp.where(c1, jv, i1)
            return nm1, nm2, nm3, nm4, ni1, ni2, ni3, ni4

        _, _, _, _, _, i2, i3, i4 = lax.fori_loop(
            0, A, col_body, (inf, inf, inf, inf, zi, zi, zi, zi))

        selfrows = (row0 % A) + rows
        obase = rows * CW
        for c in range(3):
            sv = plsc.load_gather(vblk, [selfrows * 3 + c])
            nb = (plsc.load_gather(vblk, [i2 * 3 + c])
                  + plsc.load_gather(vblk, [i3 * 3 + c])
                  + plsc.load_gather(vblk, [i4 * 3 + c]))
            plsc.store_scatter(oblk, [obase + c], sv)
            plsc.store_scatter(oblk, [obase + 3 + c], sv - nb * (1.0 / 3.0))
        plsc.store_scatter(oblk, [obase + 6], zero)
        plsc.store_scatter(oblk, [obase + 7], zero)

    pltpu.sync_copy(oblk, out_hbm.at[pl.ds(row0 * CW, RPW * CW)])


def _mlp_body(c_ref, w1_ref, b1_ref, g_ref, bt_ref, w2_ref, b2_ref, o_ref):
    cblk = c_ref[...]
    h = lax.dot_general(cblk, w1_ref[...], (((1,), (1,)), ((), ())),
                        preferred_element_type=jnp.float32)
    h = jnp.maximum(h + b1_ref[...], 0.0)
    mu = jnp.mean(h, axis=1, keepdims=True)
    xc = h - mu
    var = jnp.mean(xc * xc, axis=1, keepdims=True)
    h = xc * lax.rsqrt(var + 1e-5) * g_ref[...] + bt_ref[...]
    o_ref[...] = lax.dot_general(h, w2_ref[...], (((1,), (1,)), ((), ())),
                                 preferred_element_type=jnp.float32) + b2_ref[...]


def kernel(velocities, distance_matrix, W1, b1, gamma, beta, W2, b2):
    d2 = distance_matrix.reshape(ROWS * A)
    v2 = velocities.reshape(ROWS * 3)
    comb = _sc_neighbors(d2, v2).reshape(ROWS, CW)
    w1p = jnp.zeros((D, CW), jnp.float32).at[:, :6].set(W1)
    out = pl.pallas_call(
        _mlp_body,
        grid=(ROWS // MR,),
        in_specs=[
            pl.BlockSpec((MR, CW), lambda i: (i, 0)),
            pl.BlockSpec((D, CW), lambda i: (0, 0)),
            pl.BlockSpec((1, D), lambda i: (0, 0)),
            pl.BlockSpec((1, D), lambda i: (0, 0)),
            pl.BlockSpec((1, D), lambda i: (0, 0)),
            pl.BlockSpec((D, D), lambda i: (0, 0)),
            pl.BlockSpec((1, D), lambda i: (0, 0)),
        ],
        out_specs=pl.BlockSpec((MR, D), lambda i: (i, 0)),
        out_shape=jax.ShapeDtypeStruct((ROWS, D), jnp.float32),
    )(comb, w1p, b1.reshape(1, D), gamma.reshape(1, D), beta.reshape(1, D),
      W2, b2.reshape(1, D))
    return out.reshape(B, A, D)


# speedup vs baseline: 2.2560x; 1.0171x over previous
import functools

import jax
import jax.numpy as jnp
from jax import lax
from jax.experimental import pallas as pl
from jax.experimental.pallas import tpu as pltpu
from jax.experimental.pallas import tpu_sc as plsc

B, A, D = 16, 128, 512
NC, NS, L = 2, 16, 16
NW = NC * NS
ROWS = B * A
RPW = ROWS // NW
NG = RPW // L
CW = 8
MR = 256


@functools.partial(
    pl.kernel,
    out_type=jax.ShapeDtypeStruct((ROWS, CW), jnp.float32),
    mesh=plsc.VectorSubcoreMesh(core_axis_name="c", subcore_axis_name="s"),
    compiler_params=pltpu.CompilerParams(needs_layout_passes=False),
    scratch_types=[
        pltpu.VMEM((RPW, A), jnp.float32),
        pltpu.VMEM((A, 3), jnp.float32),
        pltpu.VMEM((RPW, CW), jnp.float32),
    ],
)
def _sc_neighbors(d_hbm, v_hbm, out_hbm, dblk, vblk, oblk):
    wid = lax.axis_index("s") * NC + lax.axis_index("c")
    row0 = wid * RPW
    b = row0 // A
    pltpu.sync_copy(d_hbm.at[pl.ds(row0, RPW), :], dblk)
    pltpu.sync_copy(v_hbm.at[pl.ds(b * A, A), :], vblk)

    iota = lax.iota(jnp.int32, L)
    inf = jnp.full((L,), jnp.inf, jnp.float32)
    zi = jnp.zeros((L,), jnp.int32)
    zero = jnp.zeros((L,), jnp.float32)

    for g in range(NG):
        rows = g * L + iota

        def col_body(j, carry, rows=rows):
            m1, m2, m3, m4, i1, i2, i3, i4 = carry
            jv = jnp.full((L,), j, jnp.int32)
            dj = plsc.load_gather(dblk, [rows, jv])
            c1 = dj < m1; c2 = dj < m2; c3 = dj < m3; c4 = dj < m4
            nm4 = jnp.where(c4, jnp.where(c3, m3, dj), m4)
            ni4 = jnp.where(c4, jnp.where(c3, i3, jv), i4)
            nm3 = jnp.where(c3, jnp.where(c2, m2, dj), m3)
            ni3 = jnp.where(c3, jnp.where(c2, i2, jv), i3)
            nm2 = jnp.where(c2, jnp.where(c1, m1, dj), m2)
            ni2 = jnp.where(c2, jnp.where(c1, i1, jv), i2)
            nm1 = jnp.where(c1, dj, m1)
            ni1 = jnp.where(c1, jv, i1)
            return nm1, nm2, nm3, nm4, ni1, ni2, ni3, ni4

        _, _, _, _, _, i2, i3, i4 = lax.fori_loop(
            0, A, col_body, (inf, inf, inf, inf, zi, zi, zi, zi))

        selfrows = (row0 % A) + rows
        for c in range(3):
            cc = jnp.full((L,), c, jnp.int32)
            sv = plsc.load_gather(vblk, [selfrows, cc])
            nb = (plsc.load_gather(vblk, [i2, cc])
                  + plsc.load_gather(vblk, [i3, cc])
                  + plsc.load_gather(vblk, [i4, cc]))
            plsc.store_scatter(oblk, [rows, cc], sv)
            plsc.store_scatter(oblk, [rows, cc + 3], sv - nb * (1.0 / 3.0))
        plsc.store_scatter(oblk, [rows, jnp.full((L,), 6, jnp.int32)], zero)
        plsc.store_scatter(oblk, [rows, jnp.full((L,), 7, jnp.int32)], zero)

    pltpu.sync_copy(oblk, out_hbm.at[pl.ds(row0, RPW), :])


def _mlp_body(c_ref, w1_ref, b1_ref, g_ref, bt_ref, w2_ref, b2_ref, o_ref):
    cblk = c_ref[...]
    h = lax.dot_general(cblk, w1_ref[...], (((1,), (1,)), ((), ())),
                        preferred_element_type=jnp.float32)
    h = jnp.maximum(h + b1_ref[...], 0.0)
    mu = jnp.mean(h, axis=1, keepdims=True)
    xc = h - mu
    var = jnp.mean(xc * xc, axis=1, keepdims=True)
    h = xc * lax.rsqrt(var + 1e-5) * g_ref[...] + bt_ref[...]
    o_ref[...] = lax.dot_general(h, w2_ref[...], (((1,), (1,)), ((), ())),
                                 preferred_element_type=jnp.float32) + b2_ref[...]


def kernel(velocities, distance_matrix, W1, b1, gamma, beta, W2, b2):
    d2 = distance_matrix.reshape(ROWS, A)
    v2 = velocities.reshape(ROWS, 3)
    comb = _sc_neighbors(d2, v2)
    w1p = jnp.zeros((D, CW), jnp.float32).at[:, :6].set(W1)
    out = pl.pallas_call(
        _mlp_body,
        grid=(ROWS // MR,),
        in_specs=[
            pl.BlockSpec((MR, CW), lambda i: (i, 0)),
            pl.BlockSpec((D, CW), lambda i: (0, 0)),
            pl.BlockSpec((1, D), lambda i: (0, 0)),
            pl.BlockSpec((1, D), lambda i: (0, 0)),
            pl.BlockSpec((1, D), lambda i: (0, 0)),
            pl.BlockSpec((D, D), lambda i: (0, 0)),
            pl.BlockSpec((1, D), lambda i: (0, 0)),
        ],
        out_specs=pl.BlockSpec((MR, D), lambda i: (i, 0)),
        out_shape=jax.ShapeDtypeStruct((ROWS, D), jnp.float32),
    )(comb, w1p, b1.reshape(1, D), gamma.reshape(1, D), beta.reshape(1, D),
      W2, b2.reshape(1, D))
    return out.reshape(B, A, D)


# speedup vs baseline: 2.3474x; 1.0405x over previous
import functools

import jax
import jax.numpy as jnp
from jax import lax
from jax.experimental import pallas as pl
from jax.experimental.pallas import tpu as pltpu
from jax.experimental.pallas import tpu_sc as plsc

B, A, D = 16, 128, 512
NC, NS, L = 2, 16, 16
NW = NC * NS
ROWS = B * A
RPW = ROWS // NW
NG = RPW // L
CW = 6
MR = 256


@functools.partial(
    pl.kernel,
    out_type=jax.ShapeDtypeStruct((ROWS, A), jnp.float32),
    mesh=plsc.VectorSubcoreMesh(core_axis_name="c", subcore_axis_name="s"),
    compiler_params=pltpu.CompilerParams(needs_layout_passes=False),
    scratch_types=[
        pltpu.VMEM((RPW, A), jnp.float32),
        pltpu.VMEM((A * 3,), jnp.float32),
        pltpu.VMEM((RPW, A), jnp.float32),
    ],
)
def _sc_neighbors(d_hbm, v_hbm, out_hbm, dblk, vblk, oblk):
    wid = lax.axis_index("s") * NC + lax.axis_index("c")
    row0 = wid * RPW
    b = row0 // A
    pltpu.sync_copy(d_hbm.at[pl.ds(row0, RPW), :], dblk)
    pltpu.sync_copy(v_hbm.at[pl.ds(b * A * 3, A * 3)], vblk)

    iota = lax.iota(jnp.int32, L)
    inf = jnp.full((L,), jnp.inf, jnp.float32)
    zi = jnp.zeros((L,), jnp.int32)

    for g in range(NG):
        rows = g * L + iota

        def col_body(j, carry, rows=rows):
            m1, m2, m3, m4, i1, i2, i3, i4 = carry
            jv = jnp.full((L,), j, jnp.int32)
            dj = plsc.load_gather(dblk, [rows, jv])
            c1 = dj < m1; c2 = dj < m2; c3 = dj < m3; c4 = dj < m4
            nm4 = jnp.where(c4, jnp.where(c3, m3, dj), m4)
            ni4 = jnp.where(c4, jnp.where(c3, i3, jv), i4)
            nm3 = jnp.where(c3, jnp.where(c2, m2, dj), m3)
            ni3 = jnp.where(c3, jnp.where(c2, i2, jv), i3)
            nm2 = jnp.where(c2, jnp.where(c1, m1, dj), m2)
            ni2 = jnp.where(c2, jnp.where(c1, i1, jv), i2)
            nm1 = jnp.where(c1, dj, m1)
            ni1 = jnp.where(c1, jv, i1)
            return nm1, nm2, nm3, nm4, ni1, ni2, ni3, ni4

        _, _, _, _, _, i2, i3, i4 = lax.fori_loop(
            0, A, col_body, (inf, inf, inf, inf, zi, zi, zi, zi))

        selfrows = (row0 % A) + rows
        for c in range(3):
            sv = plsc.load_gather(vblk, [selfrows * 3 + c])
            nb = (plsc.load_gather(vblk, [i2 * 3 + c])
                  + plsc.load_gather(vblk, [i3 * 3 + c])
                  + plsc.load_gather(vblk, [i4 * 3 + c]))
            plsc.store_scatter(oblk, [rows, jnp.full((L,), c, jnp.int32)], sv)
            plsc.store_scatter(oblk, [rows, jnp.full((L,), c + 3, jnp.int32)],
                               sv - nb * (1.0 / 3.0))

    pltpu.sync_copy(oblk, out_hbm.at[pl.ds(row0, RPW), :])


def _mlp_body(c_ref, w1_ref, b1_ref, g_ref, bt_ref, w2_ref, b2_ref, o_ref):
    cblk = c_ref[:, :CW]
    h = lax.dot_general(cblk, w1_ref[...], (((1,), (1,)), ((), ())),
                        preferred_element_type=jnp.float32)
    h = jnp.maximum(h + b1_ref[...], 0.0)
    mu = jnp.mean(h, axis=1, keepdims=True)
    xc = h - mu
    var = jnp.mean(xc * xc, axis=1, keepdims=True)
    h = xc * lax.rsqrt(var + 1e-5) * g_ref[...] + bt_ref[...]
    o_ref[...] = lax.dot_general(h, w2_ref[...], (((1,), (1,)), ((), ())),
                                 preferred_element_type=jnp.float32) + b2_ref[...]


def kernel(velocities, distance_matrix, W1, b1, gamma, beta, W2, b2):
    d2 = distance_matrix.reshape(ROWS, A)
    v2 = velocities.reshape(ROWS * 3)
    comb = _sc_neighbors(d2, v2)
    out = pl.pallas_call(
        _mlp_body,
        grid=(ROWS // MR,),
        in_specs=[
            pl.BlockSpec((MR, A), lambda i: (i, 0)),
            pl.BlockSpec((D, CW), lambda i: (0, 0)),
            pl.BlockSpec((1, D), lambda i: (0, 0)),
            pl.BlockSpec((1, D), lambda i: (0, 0)),
            pl.BlockSpec((1, D), lambda i: (0, 0)),
            pl.BlockSpec((D, D), lambda i: (0, 0)),
            pl.BlockSpec((1, D), lambda i: (0, 0)),
        ],
        out_specs=pl.BlockSpec((MR, D), lambda i: (i, 0)),
        out_shape=jax.ShapeDtypeStruct((ROWS, D), jnp.float32),
    )(comb, W1, b1.reshape(1, D), gamma.reshape(1, D), beta.reshape(1, D),
      W2, b2.reshape(1, D))
    return out.reshape(B, A, D)
